# baseline (speedup 1.0000x reference)
import jax
import jax.numpy as jnp
from jax.experimental import pallas as pl
from jax.experimental.pallas import tpu as pltpu

_E = 64
_TOP_K = 8
_D = 1024
_I = 768


def _moe_body(x_ref, gate_ref, w1_ref, w2_ref, w3_ref,
              out_ref, logits_ref, combine_ref):
    e = pl.program_id(0)
    x = x_ref[...]

    @pl.when(e == 0)
    def _router():
        logits = jax.lax.dot_general(
            x, gate_ref[...], (((1,), (1,)), ((), ())),
            preferred_element_type=jnp.float32)
        logits_ref[...] = logits
        m = jnp.max(logits, axis=1, keepdims=True)
        ex = jnp.exp(logits - m)
        probs = ex / jnp.sum(ex, axis=1, keepdims=True)
        col = jax.lax.broadcasted_iota(jnp.int32, probs.shape, 1)
        masked = probs
        comb = jnp.zeros_like(probs)
        for _ in range(_TOP_K):
            maxv = jnp.max(masked, axis=1, keepdims=True)
            idx = jnp.min(jnp.where(masked == maxv, col, _E), axis=1,
                          keepdims=True)
            onehot = col == idx
            comb = comb + jnp.where(onehot, maxv, 0.0)
            masked = jnp.where(onehot, -1.0, masked)
        comb = comb / jnp.sum(comb, axis=1, keepdims=True)
        combine_ref[...] = comb

    xb = x.astype(jnp.bfloat16)
    w1 = w1_ref[0].astype(jnp.bfloat16)
    w3 = w3_ref[0].astype(jnp.bfloat16)
    w2 = w2_ref[0].astype(jnp.bfloat16)
    g = jax.lax.dot_general(xb, w1, (((1,), (1,)), ((), ())),
                            preferred_element_type=jnp.float32)
    u = jax.lax.dot_general(xb, w3, (((1,), (1,)), ((), ())),
                            preferred_element_type=jnp.float32)
    h = (g * jax.lax.logistic(g)) * u
    y = jax.lax.dot_general(h.astype(jnp.bfloat16), w2,
                            (((1,), (1,)), ((), ())),
                            preferred_element_type=jnp.float32)
    col = jax.lax.broadcasted_iota(jnp.int32, combine_ref.shape, 1)
    c = jnp.sum(jnp.where(col == e, combine_ref[...], 0.0), axis=1,
                keepdims=True)

    @pl.when(e == 0)
    def _first():
        out_ref[...] = c * y

    @pl.when(e > 0)
    def _acc():
        out_ref[...] += c * y


def kernel(hidden_states, gate_w, w1, w2, w3):
    b, s, d = hidden_states.shape
    x = hidden_states.reshape(-1, d)
    t = x.shape[0]
    out, logits = pl.pallas_call(
        _moe_body,
        grid=(_E,),
        in_specs=[
            pl.BlockSpec((t, _D), lambda e: (0, 0)),
            pl.BlockSpec((_E, _D), lambda e: (0, 0)),
            pl.BlockSpec((1, _I, _D), lambda e: (e, 0, 0)),
            pl.BlockSpec((1, _D, _I), lambda e: (e, 0, 0)),
            pl.BlockSpec((1, _I, _D), lambda e: (e, 0, 0)),
        ],
        out_specs=[
            pl.BlockSpec((t, _D), lambda e: (0, 0)),
            pl.BlockSpec((t, _E), lambda e: (0, 0)),
        ],
        out_shape=[
            jax.ShapeDtypeStruct((t, _D), jnp.float32),
            jax.ShapeDtypeStruct((t, _E), jnp.float32),
        ],
        scratch_shapes=[pltpu.VMEM((t, _E), jnp.float32)],
        compiler_params=pltpu.CompilerParams(
            dimension_semantics=("arbitrary",)),
    )(x, gate_w, w1, w2, w3)
    return out.reshape(b, s, d), logits


# speedup vs baseline: 1.0429x; 1.0429x over previous
import jax
import jax.numpy as jnp
from jax.experimental import pallas as pl
from jax.experimental.pallas import tpu as pltpu

_E = 64
_TOP_K = 8
_D = 1024
_I = 768
_NBUF = 4


def _moe_body(x_ref, gate_ref, w1_hbm, w2_hbm, w3_hbm,
              out_ref, logits_ref,
              w1_buf, w2_buf, w3_buf, combine_ref, sems):

    def start(e):
        slot = e % _NBUF
        pltpu.make_async_copy(w1_hbm.at[e], w1_buf.at[slot],
                              sems.at[0, slot]).start()
        pltpu.make_async_copy(w2_hbm.at[e], w2_buf.at[slot],
                              sems.at[1, slot]).start()
        pltpu.make_async_copy(w3_hbm.at[e], w3_buf.at[slot],
                              sems.at[2, slot]).start()

    for e in range(_NBUF):
        start(e)

    x = x_ref[...]
    xb = x.astype(jnp.bfloat16)

    logits = jax.lax.dot_general(
        x, gate_ref[...], (((1,), (1,)), ((), ())),
        preferred_element_type=jnp.float32)
    logits_ref[...] = logits
    m = jnp.max(logits, axis=1, keepdims=True)
    ex = jnp.exp(logits - m)
    probs = ex / jnp.sum(ex, axis=1, keepdims=True)
    col = jax.lax.broadcasted_iota(jnp.int32, probs.shape, 1)
    masked = probs
    comb = jnp.zeros_like(probs)
    for _ in range(_TOP_K):
        maxv = jnp.max(masked, axis=1, keepdims=True)
        idx = jnp.min(jnp.where(masked == maxv, col, _E), axis=1,
                      keepdims=True)
        onehot = col == idx
        comb = comb + jnp.where(onehot, maxv, 0.0)
        masked = jnp.where(onehot, -1.0, masked)
    comb = comb / jnp.sum(comb, axis=1, keepdims=True)
    combine_ref[...] = comb

    def loop(e, _):
        slot = e % _NBUF
        pltpu.make_async_copy(w1_hbm.at[e], w1_buf.at[slot],
                              sems.at[0, slot]).wait()
        pltpu.make_async_copy(w2_hbm.at[e], w2_buf.at[slot],
                              sems.at[1, slot]).wait()
        pltpu.make_async_copy(w3_hbm.at[e], w3_buf.at[slot],
                              sems.at[2, slot]).wait()
        w1 = w1_buf[slot].astype(jnp.bfloat16)
        w3 = w3_buf[slot].astype(jnp.bfloat16)
        w2 = w2_buf[slot].astype(jnp.bfloat16)
        g = jax.lax.dot_general(xb, w1, (((1,), (1,)), ((), ())),
                                preferred_element_type=jnp.float32)
        u = jax.lax.dot_general(xb, w3, (((1,), (1,)), ((), ())),
                                preferred_element_type=jnp.float32)
        h = (g * jax.lax.logistic(g)) * u
        y = jax.lax.dot_general(h.astype(jnp.bfloat16), w2,
                                (((1,), (1,)), ((), ())),
                                preferred_element_type=jnp.float32)

        @pl.when(e + _NBUF < _E)
        def _prefetch():
            start(e + _NBUF)

        c = jnp.sum(jnp.where(col == e, combine_ref[...], 0.0), axis=1,
                    keepdims=True)

        @pl.when(e == 0)
        def _first():
            out_ref[...] = c * y

        @pl.when(e > 0)
        def _acc():
            out_ref[...] += c * y
        return 0

    jax.lax.fori_loop(0, _E, loop, 0)


def kernel(hidden_states, gate_w, w1, w2, w3):
    b, s, d = hidden_states.shape
    x = hidden_states.reshape(-1, d)
    t = x.shape[0]
    out, logits = pl.pallas_call(
        _moe_body,
        in_specs=[
            pl.BlockSpec(memory_space=pltpu.VMEM),
            pl.BlockSpec(memory_space=pltpu.VMEM),
            pl.BlockSpec(memory_space=pl.ANY),
            pl.BlockSpec(memory_space=pl.ANY),
            pl.BlockSpec(memory_space=pl.ANY),
        ],
        out_specs=[
            pl.BlockSpec(memory_space=pltpu.VMEM),
            pl.BlockSpec(memory_space=pltpu.VMEM),
        ],
        out_shape=[
            jax.ShapeDtypeStruct((t, _D), jnp.float32),
            jax.ShapeDtypeStruct((t, _E), jnp.float32),
        ],
        scratch_shapes=[
            pltpu.VMEM((_NBUF, _I, _D), jnp.float32),
            pltpu.VMEM((_NBUF, _D, _I), jnp.float32),
            pltpu.VMEM((_NBUF, _I, _D), jnp.float32),
            pltpu.VMEM((t, _E), jnp.float32),
            pltpu.SemaphoreType.DMA((3, _NBUF)),
        ],
    )(x, gate_w, w1, w2, w3)
    return out.reshape(b, s, d), logits
